# baseline (speedup 1.0000x reference)
import functools

import jax
import jax.numpy as jnp
from jax import lax
from jax.experimental import pallas as pl
from jax.experimental.pallas import tpu as pltpu
from jax.experimental.pallas import tpu_sc as plsc

ENC_DIM = 1280
K = 2
IN_DIM = ENC_DIM * K
OUT_DIM = 4096
NUM_EXPERTS = 8
TOP_K = 4
ROUTED_HIDDEN = 2048
SHARED_HIDDEN = 2048
EPS = 1e-6
NT = 4096
LANES = 128

TT = 256
NTILE = 72
CAP = NTILE * TT
NA = NT * TOP_K
NW = 32
TPW = NT // NW
APW = TPW * TOP_K
CH = 32
NCH = APW // CH


def _norm_body(x_ref, w_ref, rw_ref, nx_ref, lg_ref):
    x = x_ref[...]
    var = jnp.mean(x * x, axis=-1, keepdims=True)
    nx = x * jax.lax.rsqrt(var + EPS) * w_ref[...]
    nx_ref[...] = nx.astype(jnp.bfloat16)
    lg_ref[...] = jax.lax.dot_general(
        nx, rw_ref[...], (((1,), (1,)), ((), ())),
        preferred_element_type=jnp.float32)


def _k_norm(tokens, ln_pre_w, router_w_pad):
    rt = 16
    bt = NT // rt
    return pl.pallas_call(
        _norm_body,
        grid=(rt,),
        in_specs=[
            pl.BlockSpec((bt, IN_DIM), lambda i: (i, 0)),
            pl.BlockSpec((1, IN_DIM), lambda i: (0, 0)),
            pl.BlockSpec((LANES, IN_DIM), lambda i: (0, 0)),
        ],
        out_specs=[
            pl.BlockSpec((bt, IN_DIM), lambda i: (i, 0)),
            pl.BlockSpec((bt, LANES), lambda i: (i, 0)),
        ],
        out_shape=[
            jax.ShapeDtypeStruct((NT, IN_DIM), jnp.bfloat16),
            jax.ShapeDtypeStruct((NT, LANES), jnp.float32),
        ],
    )(tokens, ln_pre_w.reshape(1, IN_DIM), router_w_pad)


def _cumsum_lanes(x, width):
    k = 1
    while k < width:
        pad = jnp.zeros_like(x[:, :k])
        x = x + jnp.concatenate([pad, x[:, :-k]], axis=1)
        k *= 2
    return x


def _cumsum_rows(x):
    k = 1
    while k < x.shape[0]:
        pad = jnp.zeros_like(x[:k, :])
        x = x + jnp.concatenate([pad, x[:-k, :]], axis=0)
        k *= 2
    return x


def _route_body(lg_ref, q4_ref, wq4_ref, te_ref, aux_ref):
    lg = lg_ref[...]
    lane = jax.lax.broadcasted_iota(jnp.int32, lg.shape, 1)
    valid = lane < NUM_EXPERTS
    neg = jnp.float32(-1e30)
    lg = jnp.where(valid, lg, neg)
    m = jnp.max(lg, axis=-1, keepdims=True)
    e = jnp.where(valid, jnp.exp(lg - m), 0.0)
    p = e / jnp.sum(e, axis=-1, keepdims=True)

    imp = jnp.sum(p, axis=0)
    aux = jnp.sum(imp * imp) / (NT * NT) * NUM_EXPERTS
    aux_ref[0, 0] = aux

    work = p
    w8 = jnp.zeros_like(p)
    selmask = jnp.zeros(p.shape, jnp.int32)
    wsum = jnp.zeros((p.shape[0], 1), jnp.float32)
    picks = []
    for _ in range(TOP_K):
        cur = jnp.max(work, axis=-1, keepdims=True)
        idx = jnp.min(jnp.where(work == cur, lane, LANES), axis=-1,
                      keepdims=True)
        picks.append((idx, cur))
        wsum = wsum + cur
        work = jnp.where(lane == idx, neg, work)
        selmask = selmask + jnp.where(lane == idx, 1, 0)
    inv = 1.0 / (wsum + 1e-20)
    for idx, cur in picks:
        w8 = w8 + jnp.where(lane == idx, cur * inv, 0.0)

    n_e = jnp.sum(selmask, axis=0, keepdims=True)
    rounded = (((n_e + TT - 1) >> 8) << 8)
    off = _cumsum_lanes(rounded, NUM_EXPERTS) - rounded
    erank = _cumsum_rows(selmask) - selmask
    slot = off + erank
    jrank = _cumsum_lanes(selmask, NUM_EXPERTS) - selmask
    sel = selmask > 0
    qcols, wcols = [], []
    for j in range(TOP_K):
        pickj = sel & (jrank == j)
        qcols.append(jnp.sum(jnp.where(pickj, slot, 0), axis=-1,
                             keepdims=True))
        wcols.append(jnp.sum(jnp.where(pickj, w8, 0.0), axis=-1,
                             keepdims=True))
    q4_ref[...] = jnp.concatenate(qcols, axis=-1)
    wq4_ref[...] = jnp.concatenate(wcols, axis=-1)

    kk = jax.lax.broadcasted_iota(jnp.int32, (LANES, LANES), 0) * TT
    ee = jax.lax.broadcasted_iota(jnp.int32, (LANES, LANES), 1)
    cmp = jnp.where((kk >= off[0:1, :]) & (ee < NUM_EXPERTS), 1, 0)
    te_ref[...] = jnp.sum(cmp, axis=-1, keepdims=True) - 1


def _k_route(logits):
    return pl.pallas_call(
        _route_body,
        out_shape=[
            jax.ShapeDtypeStruct((NT, TOP_K), jnp.int32),
            jax.ShapeDtypeStruct((NT, TOP_K), jnp.float32),
            jax.ShapeDtypeStruct((LANES, 1), jnp.int32),
            jax.ShapeDtypeStruct((1, 1), jnp.float32),
        ],
        out_specs=[
            pl.BlockSpec((NT, TOP_K), lambda: (0, 0)),
            pl.BlockSpec((NT, TOP_K), lambda: (0, 0)),
            pl.BlockSpec((LANES, 1), lambda: (0, 0)),
            pl.BlockSpec(memory_space=pltpu.SMEM),
        ],
    )(logits)


_NC16 = APW // 16


def _sc_disp_body(q_hbm, nx_hbm, disp_hbm, qv, rows0, rows1, semg, sems):
    wid = lax.axis_index("s") * 2 + lax.axis_index("c")
    lane = jax.lax.broadcasted_iota(jnp.int32, (16,), 0)
    pltpu.sync_copy(q_hbm.at[wid], qv)
    bufs = (rows0, rows1)

    def gather(c):
        gvec = (wid * APW + c * 16 + lane) >> 2
        return pltpu.async_copy(nx_hbm.at[gvec], bufs[c % 2], semg)

    g = gather(0)
    for c in range(_NC16):
        g_next = gather(c + 1) if c + 1 < _NC16 else None
        g.wait()
        svec = qv[pl.ds(c * 16, 16)]
        pltpu.async_copy(bufs[c % 2], disp_hbm.at[svec], sems).wait()
        g = g_next


def _sc_dispatch(q_flat, nx_i32):
    mesh = plsc.VectorSubcoreMesh(core_axis_name="c", subcore_axis_name="s")
    f = functools.partial(
        pl.kernel,
        out_type=jax.ShapeDtypeStruct((CAP, ENC_DIM), jnp.int32),
        mesh=mesh,
        scratch_types=[
            pltpu.VMEM((APW,), jnp.int32),
            pltpu.VMEM((16, ENC_DIM), jnp.int32),
            pltpu.VMEM((16, ENC_DIM), jnp.int32),
            pltpu.SemaphoreType.DMA,
            pltpu.SemaphoreType.DMA,
        ],
    )(_sc_disp_body)
    return f(q_flat, nx_i32)


def _sc_g2_body(dout_hbm, q_hbm, g_hbm, qv, rows0, rows1, semg):
    wid = lax.axis_index("s") * 2 + lax.axis_index("c")
    pltpu.sync_copy(q_hbm.at[wid], qv)
    bufs = (rows0, rows1)

    def gather(c):
        svec = qv[pl.ds(c * 16, 16)]
        return pltpu.async_copy(dout_hbm.at[svec], bufs[c % 2], semg)

    g = gather(0)
    for c in range(_NC16):
        g_next = gather(c + 1) if c + 1 < _NC16 else None
        g.wait()
        pltpu.sync_copy(bufs[c % 2], g_hbm.at[pl.ds(wid * APW + c * 16, 16)])
        g = g_next


def _sc_g2(dout_i32, q_flat):
    mesh = plsc.VectorSubcoreMesh(core_axis_name="c", subcore_axis_name="s")
    f = functools.partial(
        pl.kernel,
        out_type=jax.ShapeDtypeStruct((NA, OUT_DIM // 2), jnp.int32),
        mesh=mesh,
        scratch_types=[
            pltpu.VMEM((APW,), jnp.int32),
            pltpu.VMEM((16, OUT_DIM // 2), jnp.int32),
            pltpu.VMEM((16, OUT_DIM // 2), jnp.int32),
            pltpu.SemaphoreType.DMA,
        ],
    )(_sc_g2_body)
    return f(dout_i32, q_flat)


def _k1_body(te_ref, nx_ref, wg_ref, wv_ref, h_ref):
    nx = nx_ref[...]
    g = jax.lax.dot_general(nx, wg_ref[0], (((1,), (1,)), ((), ())),
                            preferred_element_type=jnp.float32)
    v = jax.lax.dot_general(nx, wv_ref[0], (((1,), (1,)), ((), ())),
                            preferred_element_type=jnp.float32)
    h_ref[...] = (g * jax.lax.logistic(g) * v).astype(jnp.bfloat16)


def _k_gmm1(te, disp_bf16, ew12):
    bh = 1024
    nh = ROUTED_HIDDEN // bh
    return pl.pallas_call(
        _k1_body,
        grid_spec=pltpu.PrefetchScalarGridSpec(
            num_scalar_prefetch=1,
            grid=(nh, NTILE),
            in_specs=[
                pl.BlockSpec((TT, IN_DIM), lambda c, i, te: (i, 0)),
                pl.BlockSpec((1, bh, IN_DIM), lambda c, i, te: (te[i], c, 0)),
                pl.BlockSpec((1, bh, IN_DIM),
                             lambda c, i, te: (te[i], nh + c, 0)),
            ],
            out_specs=pl.BlockSpec((TT, bh), lambda c, i, te: (i, c)),
        ),
        out_shape=jax.ShapeDtypeStruct((CAP, ROUTED_HIDDEN), jnp.bfloat16),
    )(te, disp_bf16, ew12, ew12)


def _k2_body(te_ref, h_ref, w3_ref, o_ref):
    o_ref[...] = jax.lax.dot_general(
        h_ref[...], w3_ref[0], (((1,), (1,)), ((), ())),
        preferred_element_type=jnp.float32).astype(jnp.bfloat16)


def _k_gmm2(te, hb, ew3):
    bo = 1024
    return pl.pallas_call(
        _k2_body,
        grid_spec=pltpu.PrefetchScalarGridSpec(
            num_scalar_prefetch=1,
            grid=(OUT_DIM // bo, NTILE),
            in_specs=[
                pl.BlockSpec((TT, ROUTED_HIDDEN), lambda c, i, te: (i, 0)),
                pl.BlockSpec((1, bo, ROUTED_HIDDEN),
                             lambda c, i, te: (te[i], c, 0)),
            ],
            out_specs=pl.BlockSpec((TT, bo), lambda c, i, te: (i, c)),
        ),
        out_shape=jax.ShapeDtypeStruct((CAP, OUT_DIM), jnp.bfloat16),
    )(te, hb, ew3)


def _s1_body(nx_ref, wg_ref, wv_ref, h_ref):
    nx = nx_ref[...]
    g = jax.lax.dot_general(nx, wg_ref[...], (((1,), (1,)), ((), ())),
                            preferred_element_type=jnp.float32)
    v = jax.lax.dot_general(nx, wv_ref[...], (((1,), (1,)), ((), ())),
                            preferred_element_type=jnp.float32)
    h_ref[...] = (g * jax.lax.logistic(g) * v).astype(jnp.bfloat16)


def _k_s1(nx, w12):
    rt, bh = 8, 1024
    bt = NT // rt
    return pl.pallas_call(
        _s1_body,
        grid=(SHARED_HIDDEN // bh, rt),
        in_specs=[
            pl.BlockSpec((bt, IN_DIM), lambda c, i: (i, 0)),
            pl.BlockSpec((bh, IN_DIM), lambda c, i: (c, 0)),
            pl.BlockSpec((bh, IN_DIM),
                         lambda c, i: (SHARED_HIDDEN // bh + c, 0)),
        ],
        out_specs=pl.BlockSpec((bt, bh), lambda c, i: (i, c)),
        out_shape=jax.ShapeDtypeStruct((NT, SHARED_HIDDEN), jnp.bfloat16),
    )(nx, w12, w12)


def _final_body(g_ref, wq_ref, hs_ref, w3_ref, wpost_ref, o_ref):
    sh = jax.lax.dot_general(
        hs_ref[...], w3_ref[...], (((1,), (1,)), ((), ())),
        preferred_element_type=jnp.float32)
    bt = sh.shape[0]
    g3 = g_ref[...].reshape(bt, TOP_K, OUT_DIM).astype(jnp.float32)
    wq = wq_ref[...]
    y = sh
    for j in range(TOP_K):
        y = y + wq[:, j:j + 1] * g3[:, j, :]
    var = jnp.mean(y * y, axis=-1, keepdims=True)
    y = y * jax.lax.rsqrt(var + EPS) * wpost_ref[...]
    o_ref[...] = jnp.clip(y, -30.0, 30.0)


def _k_final(g_bf16, wq4, hs, sw3, ln_post_w):
    rt = 16
    bt = NT // rt
    return pl.pallas_call(
        _final_body,
        grid=(rt,),
        in_specs=[
            pl.BlockSpec((bt * TOP_K, OUT_DIM), lambda i: (i, 0)),
            pl.BlockSpec((bt, TOP_K), lambda i: (i, 0)),
            pl.BlockSpec((bt, SHARED_HIDDEN), lambda i: (i, 0)),
            pl.BlockSpec((OUT_DIM, SHARED_HIDDEN), lambda i: (0, 0)),
            pl.BlockSpec((1, OUT_DIM), lambda i: (0, 0)),
        ],
        out_specs=pl.BlockSpec((bt, OUT_DIM), lambda i: (i, 0)),
        out_shape=jax.ShapeDtypeStruct((NT, OUT_DIM), jnp.float32),
    )(g_bf16, wq4, hs, sw3, ln_post_w.reshape(1, OUT_DIM))


def _as_i32(x):
    n = x.shape[-1] // 2
    return jax.lax.bitcast_convert_type(
        x.reshape(*x.shape[:-1], n, 2), jnp.int32)


def _as_bf16(x):
    y = jax.lax.bitcast_convert_type(x, jnp.bfloat16)
    return y.reshape(*x.shape[:-1], x.shape[-1] * 2)


def kernel(x, ln_pre_w, router_w, shared_w12, shared_w3, expert_w12,
           expert_w3, ln_post_w):
    B, S, D = x.shape
    tokens = x.reshape(B * S // K, D * K)

    router_w_pad = jnp.zeros((LANES, IN_DIM), jnp.float32).at[:NUM_EXPERTS].set(
        router_w)
    nx, logits = _k_norm(tokens, ln_pre_w, router_w_pad)
    q4, wq4, te2d, aux = _k_route(logits)
    te = te2d.reshape(LANES)[:80]
    q_flat = q4.reshape(NW, APW)

    disp_i32 = _sc_dispatch(q_flat, _as_i32(nx))

    ew12 = expert_w12.astype(jnp.bfloat16)
    ew3 = expert_w3.astype(jnp.bfloat16)
    hb = _k_gmm1(te, _as_bf16(disp_i32), ew12)
    dout = _k_gmm2(te, hb, ew3)

    g_i32 = _sc_g2(_as_i32(dout), q_flat)
    hs = _k_s1(nx, shared_w12.astype(jnp.bfloat16))
    final = _k_final(_as_bf16(g_i32), wq4, hs,
                     shared_w3.astype(jnp.bfloat16), ln_post_w)
    return final.reshape(B, S // K, OUT_DIM), aux[0, 0]


# speedup vs baseline: 2.7539x; 2.7539x over previous
import functools

import jax
import jax.numpy as jnp
from jax import lax
from jax.experimental import pallas as pl
from jax.experimental.pallas import tpu as pltpu
from jax.experimental.pallas import tpu_sc as plsc

ENC_DIM = 1280
K = 2
IN_DIM = ENC_DIM * K
OUT_DIM = 4096
NUM_EXPERTS = 8
TOP_K = 4
ROUTED_HIDDEN = 2048
SHARED_HIDDEN = 2048
EPS = 1e-6
NT = 4096
LANES = 128

TT = 256
NTILE = 72
CAP = NTILE * TT
NA = NT * TOP_K
NW = 32
TPW = NT // NW
APW = TPW * TOP_K
CH = 32
NCH = APW // CH


def _norm_body(x_ref, w_ref, rw_ref, nx_ref, lg_ref):
    x = x_ref[...]
    var = jnp.mean(x * x, axis=-1, keepdims=True)
    nx = x * jax.lax.rsqrt(var + EPS) * w_ref[...]
    nx_ref[...] = nx
    lg_ref[...] = jax.lax.dot_general(
        nx, rw_ref[...], (((1,), (1,)), ((), ())),
        preferred_element_type=jnp.float32)


def _k_norm(tokens, ln_pre_w, router_w_pad):
    rt = 16
    bt = NT // rt
    return pl.pallas_call(
        _norm_body,
        grid=(rt,),
        in_specs=[
            pl.BlockSpec((bt, IN_DIM), lambda i: (i, 0)),
            pl.BlockSpec((1, IN_DIM), lambda i: (0, 0)),
            pl.BlockSpec((LANES, IN_DIM), lambda i: (0, 0)),
        ],
        out_specs=[
            pl.BlockSpec((bt, IN_DIM), lambda i: (i, 0)),
            pl.BlockSpec((bt, LANES), lambda i: (i, 0)),
        ],
        out_shape=[
            jax.ShapeDtypeStruct((NT, IN_DIM), jnp.float32),
            jax.ShapeDtypeStruct((NT, LANES), jnp.float32),
        ],
    )(tokens, ln_pre_w.reshape(1, IN_DIM), router_w_pad)


def _cumsum_lanes(x, width):
    k = 1
    while k < width:
        pad = jnp.zeros_like(x[:, :k])
        x = x + jnp.concatenate([pad, x[:, :-k]], axis=1)
        k *= 2
    return x


def _cumsum_rows(x):
    k = 1
    while k < x.shape[0]:
        pad = jnp.zeros_like(x[:k, :])
        x = x + jnp.concatenate([pad, x[:-k, :]], axis=0)
        k *= 2
    return x


def _route_body(lg_ref, q4_ref, wq4_ref, te_ref, aux_ref):
    lg = lg_ref[...]
    lane = jax.lax.broadcasted_iota(jnp.int32, lg.shape, 1)
    valid = lane < NUM_EXPERTS
    neg = jnp.float32(-1e30)
    lg = jnp.where(valid, lg, neg)
    m = jnp.max(lg, axis=-1, keepdims=True)
    e = jnp.where(valid, jnp.exp(lg - m), 0.0)
    p = e / jnp.sum(e, axis=-1, keepdims=True)

    imp = jnp.sum(p, axis=0)
    aux = jnp.sum(imp * imp) / (NT * NT) * NUM_EXPERTS
    aux_ref[0, 0] = aux

    work = p
    w8 = jnp.zeros_like(p)
    selmask = jnp.zeros(p.shape, jnp.int32)
    wsum = jnp.zeros((p.shape[0], 1), jnp.float32)
    picks = []
    for _ in range(TOP_K):
        cur = jnp.max(work, axis=-1, keepdims=True)
        idx = jnp.min(jnp.where(work == cur, lane, LANES), axis=-1,
                      keepdims=True)
        picks.append((idx, cur))
        wsum = wsum + cur
        work = jnp.where(lane == idx, neg, work)
        selmask = selmask + jnp.where(lane == idx, 1, 0)
    inv = 1.0 / (wsum + 1e-20)
    for idx, cur in picks:
        w8 = w8 + jnp.where(lane == idx, cur * inv, 0.0)

    n_e = jnp.sum(selmask, axis=0, keepdims=True)
    rounded = (((n_e + TT - 1) >> 8) << 8)
    off = _cumsum_lanes(rounded, NUM_EXPERTS) - rounded
    erank = _cumsum_rows(selmask) - selmask
    slot = off + erank
    jrank = _cumsum_lanes(selmask, NUM_EXPERTS) - selmask
    sel = selmask > 0
    qcols, wcols = [], []
    for j in range(TOP_K):
        pickj = sel & (jrank == j)
        qcols.append(jnp.sum(jnp.where(pickj, slot, 0), axis=-1,
                             keepdims=True))
        wcols.append(jnp.sum(jnp.where(pickj, w8, 0.0), axis=-1,
                             keepdims=True))
    q4_ref[...] = jnp.concatenate(qcols, axis=-1)
    wq4_ref[...] = jnp.concatenate(wcols, axis=-1)

    kk = jax.lax.broadcasted_iota(jnp.int32, (LANES, LANES), 0) * TT
    ee = jax.lax.broadcasted_iota(jnp.int32, (LANES, LANES), 1)
    cmp = jnp.where((kk >= off[0:1, :]) & (ee < NUM_EXPERTS), 1, 0)
    te_ref[...] = jnp.sum(cmp, axis=-1, keepdims=True) - 1


def _k_route(logits):
    return pl.pallas_call(
        _route_body,
        out_shape=[
            jax.ShapeDtypeStruct((NT, TOP_K), jnp.int32),
            jax.ShapeDtypeStruct((NT, TOP_K), jnp.float32),
            jax.ShapeDtypeStruct((LANES, 1), jnp.int32),
            jax.ShapeDtypeStruct((1, 1), jnp.float32),
        ],
        out_specs=[
            pl.BlockSpec((NT, TOP_K), lambda: (0, 0)),
            pl.BlockSpec((NT, TOP_K), lambda: (0, 0)),
            pl.BlockSpec((LANES, 1), lambda: (0, 0)),
            pl.BlockSpec(memory_space=pltpu.SMEM),
        ],
    )(logits)


_NC16 = APW // 16


def _sc_disp_body(q_hbm, nx_hbm, disp_hbm, qv, rows0, rows1, semg, sems):
    wid = lax.axis_index("s") * 2 + lax.axis_index("c")
    lane = jax.lax.broadcasted_iota(jnp.int32, (16,), 0)
    pltpu.sync_copy(q_hbm.at[wid], qv)
    bufs = (rows0, rows1)

    def gather(c):
        gvec = (wid * APW + c * 16 + lane) >> 2
        return pltpu.async_copy(nx_hbm.at[gvec], bufs[c % 2], semg)

    g = gather(0)
    for c in range(_NC16):
        g_next = gather(c + 1) if c + 1 < _NC16 else None
        g.wait()
        svec = qv[pl.ds(c * 16, 16)]
        pltpu.async_copy(bufs[c % 2], disp_hbm.at[svec], sems).wait()
        g = g_next


def _sc_dispatch(q_flat, nx_f32):
    mesh = plsc.VectorSubcoreMesh(core_axis_name="c", subcore_axis_name="s")
    f = functools.partial(
        pl.kernel,
        out_type=jax.ShapeDtypeStruct((CAP, IN_DIM), jnp.float32),
        mesh=mesh,
        scratch_types=[
            pltpu.VMEM((APW,), jnp.int32),
            pltpu.VMEM((16, IN_DIM), jnp.float32),
            pltpu.VMEM((16, IN_DIM), jnp.float32),
            pltpu.SemaphoreType.DMA,
            pltpu.SemaphoreType.DMA,
        ],
    )(_sc_disp_body)
    return f(q_flat, nx_f32)


_NC8 = APW // 8


def _sc_g2_body(dout_hbm, q_hbm, g_hbm, qv, rows0, rows1, semg):
    wid = lax.axis_index("s") * 2 + lax.axis_index("c")
    pltpu.sync_copy(q_hbm.at[wid], qv)
    bufs = (rows0, rows1)

    def gather(c):
        return pltpu.async_copy(
            dout_hbm.at[qv.at[pl.ds(c * 8, 8)]], bufs[c % 2], semg)

    g = gather(0)
    for c in range(_NC8):
        g_next = gather(c + 1) if c + 1 < _NC8 else None
        g.wait()
        pltpu.sync_copy(bufs[c % 2], g_hbm.at[pl.ds(wid * APW + c * 8, 8)])
        g = g_next


def _sc_g2(dout_f32, q_flat):
    mesh = plsc.VectorSubcoreMesh(core_axis_name="c", subcore_axis_name="s")
    f = functools.partial(
        pl.kernel,
        out_type=jax.ShapeDtypeStruct((NA, OUT_DIM), jnp.float32),
        mesh=mesh,
        scratch_types=[
            pltpu.VMEM((APW,), jnp.int32),
            pltpu.VMEM((8, OUT_DIM), jnp.float32),
            pltpu.VMEM((8, OUT_DIM), jnp.float32),
            pltpu.SemaphoreType.DMA,
        ],
    )(_sc_g2_body)
    return f(dout_f32, q_flat)


def _k1_body(te_ref, nx_ref, wg_ref, wv_ref, h_ref):
    nx = nx_ref[...].astype(jnp.bfloat16)
    g = jax.lax.dot_general(nx, wg_ref[0], (((1,), (1,)), ((), ())),
                            preferred_element_type=jnp.float32)
    v = jax.lax.dot_general(nx, wv_ref[0], (((1,), (1,)), ((), ())),
                            preferred_element_type=jnp.float32)
    h_ref[...] = (g * jax.lax.logistic(g) * v).astype(jnp.bfloat16)


def _k_gmm1(te, disp_bf16, ew12):
    bh = 1024
    nh = ROUTED_HIDDEN // bh
    return pl.pallas_call(
        _k1_body,
        grid_spec=pltpu.PrefetchScalarGridSpec(
            num_scalar_prefetch=1,
            grid=(nh, NTILE),
            in_specs=[
                pl.BlockSpec((TT, IN_DIM), lambda c, i, te: (i, 0)),
                pl.BlockSpec((1, bh, IN_DIM), lambda c, i, te: (te[i], c, 0)),
                pl.BlockSpec((1, bh, IN_DIM),
                             lambda c, i, te: (te[i], nh + c, 0)),
            ],
            out_specs=pl.BlockSpec((TT, bh), lambda c, i, te: (i, c)),
        ),
        out_shape=jax.ShapeDtypeStruct((CAP, ROUTED_HIDDEN), jnp.bfloat16),
    )(te, disp_bf16, ew12, ew12)


def _k2_body(te_ref, h_ref, w3_ref, o_ref):
    o_ref[...] = jax.lax.dot_general(
        h_ref[...], w3_ref[0], (((1,), (1,)), ((), ())),
        preferred_element_type=jnp.float32)


def _k_gmm2(te, hb, ew3):
    bo = 1024
    return pl.pallas_call(
        _k2_body,
        grid_spec=pltpu.PrefetchScalarGridSpec(
            num_scalar_prefetch=1,
            grid=(OUT_DIM // bo, NTILE),
            in_specs=[
                pl.BlockSpec((TT, ROUTED_HIDDEN), lambda c, i, te: (i, 0)),
                pl.BlockSpec((1, bo, ROUTED_HIDDEN),
                             lambda c, i, te: (te[i], c, 0)),
            ],
            out_specs=pl.BlockSpec((TT, bo), lambda c, i, te: (i, c)),
        ),
        out_shape=jax.ShapeDtypeStruct((CAP, OUT_DIM), jnp.float32),
    )(te, hb, ew3)


def _s1_body(nx_ref, wg_ref, wv_ref, h_ref):
    nx = nx_ref[...].astype(jnp.bfloat16)
    g = jax.lax.dot_general(nx, wg_ref[...], (((1,), (1,)), ((), ())),
                            preferred_element_type=jnp.float32)
    v = jax.lax.dot_general(nx, wv_ref[...], (((1,), (1,)), ((), ())),
                            preferred_element_type=jnp.float32)
    h_ref[...] = (g * jax.lax.logistic(g) * v).astype(jnp.bfloat16)


def _k_s1(nx, w12):
    rt, bh = 8, 1024
    bt = NT // rt
    return pl.pallas_call(
        _s1_body,
        grid=(SHARED_HIDDEN // bh, rt),
        in_specs=[
            pl.BlockSpec((bt, IN_DIM), lambda c, i: (i, 0)),
            pl.BlockSpec((bh, IN_DIM), lambda c, i: (c, 0)),
            pl.BlockSpec((bh, IN_DIM),
                         lambda c, i: (SHARED_HIDDEN // bh + c, 0)),
        ],
        out_specs=pl.BlockSpec((bt, bh), lambda c, i: (i, c)),
        out_shape=jax.ShapeDtypeStruct((NT, SHARED_HIDDEN), jnp.bfloat16),
    )(nx, w12, w12)


def _final_body(g_ref, wq_ref, hs_ref, w3_ref, wpost_ref, o_ref):
    sh = jax.lax.dot_general(
        hs_ref[...], w3_ref[...], (((1,), (1,)), ((), ())),
        preferred_element_type=jnp.float32)
    bt = sh.shape[0]
    g3 = g_ref[...].reshape(bt, TOP_K, OUT_DIM)
    wq = wq_ref[...]
    y = sh
    for j in range(TOP_K):
        y = y + wq[:, j:j + 1] * g3[:, j, :]
    var = jnp.mean(y * y, axis=-1, keepdims=True)
    y = y * jax.lax.rsqrt(var + EPS) * wpost_ref[...]
    o_ref[...] = jnp.clip(y, -30.0, 30.0)


def _k_final(g_f32, wq4, hs, sw3, ln_post_w):
    rt = 32
    bt = NT // rt
    return pl.pallas_call(
        _final_body,
        grid=(rt,),
        in_specs=[
            pl.BlockSpec((bt * TOP_K, OUT_DIM), lambda i: (i, 0)),
            pl.BlockSpec((bt, TOP_K), lambda i: (i, 0)),
            pl.BlockSpec((bt, SHARED_HIDDEN), lambda i: (i, 0)),
            pl.BlockSpec((OUT_DIM, SHARED_HIDDEN), lambda i: (0, 0)),
            pl.BlockSpec((1, OUT_DIM), lambda i: (0, 0)),
        ],
        out_specs=pl.BlockSpec((bt, OUT_DIM), lambda i: (i, 0)),
        out_shape=jax.ShapeDtypeStruct((NT, OUT_DIM), jnp.float32),
    )(g_f32, wq4, hs, sw3, ln_post_w.reshape(1, OUT_DIM))


def kernel(x, ln_pre_w, router_w, shared_w12, shared_w3, expert_w12,
           expert_w3, ln_post_w):
    B, S, D = x.shape
    tokens = x.reshape(B * S // K, D * K)

    router_w_pad = jnp.zeros((LANES, IN_DIM), jnp.float32).at[:NUM_EXPERTS].set(
        router_w)
    nx, logits = _k_norm(tokens, ln_pre_w, router_w_pad)
    q4, wq4, te2d, aux = _k_route(logits)
    te = te2d.reshape(LANES)[:80]
    q_flat = q4.reshape(NW, APW)

    disp = _sc_dispatch(q_flat, nx)

    ew12 = expert_w12.astype(jnp.bfloat16)
    ew3 = expert_w3.astype(jnp.bfloat16)
    hb = _k_gmm1(te, disp, ew12)
    dout = _k_gmm2(te, hb, ew3)

    g = _sc_g2(dout, q_flat)
    hs = _k_s1(nx, shared_w12.astype(jnp.bfloat16))
    final = _k_final(g, wq4, hs,
                     shared_w3.astype(jnp.bfloat16), ln_post_w)
    return final.reshape(B, S // K, OUT_DIM), aux[0, 0]


# speedup vs baseline: 3.0416x; 1.1045x over previous
import functools

import jax
import jax.numpy as jnp
from jax import lax
from jax.experimental import pallas as pl
from jax.experimental.pallas import tpu as pltpu
from jax.experimental.pallas import tpu_sc as plsc

ENC_DIM = 1280
K = 2
IN_DIM = ENC_DIM * K
OUT_DIM = 4096
NUM_EXPERTS = 8
TOP_K = 4
ROUTED_HIDDEN = 2048
SHARED_HIDDEN = 2048
EPS = 1e-6
NT = 4096
LANES = 128

TT = 256
NTILE = 72
CAP = NTILE * TT
NA = NT * TOP_K
NW = 32
TPW = NT // NW
APW = TPW * TOP_K


def _norm_body(x_ref, w_ref, rw_ref, nx_ref, lg_ref):
    x = x_ref[...]
    var = jnp.mean(x * x, axis=-1, keepdims=True)
    nx = x * jax.lax.rsqrt(var + EPS) * w_ref[...]
    nx_ref[...] = nx
    lg_ref[...] = jax.lax.dot_general(
        nx, rw_ref[...], (((1,), (1,)), ((), ())),
        preferred_element_type=jnp.float32)


def _k_norm(tokens, ln_pre_w, router_w_pad):
    rt = 16
    bt = NT // rt
    return pl.pallas_call(
        _norm_body,
        grid=(rt,),
        in_specs=[
            pl.BlockSpec((bt, IN_DIM), lambda i: (i, 0)),
            pl.BlockSpec((1, IN_DIM), lambda i: (0, 0)),
            pl.BlockSpec((LANES, IN_DIM), lambda i: (0, 0)),
        ],
        out_specs=[
            pl.BlockSpec((bt, IN_DIM), lambda i: (i, 0)),
            pl.BlockSpec((bt, LANES), lambda i: (i, 0)),
        ],
        out_shape=[
            jax.ShapeDtypeStruct((NT, IN_DIM), jnp.float32),
            jax.ShapeDtypeStruct((NT, LANES), jnp.float32),
        ],
    )(tokens, ln_pre_w.reshape(1, IN_DIM), router_w_pad)


def _cumsum_lanes(x, width):
    k = 1
    while k < width:
        pad = jnp.zeros_like(x[:, :k])
        x = x + jnp.concatenate([pad, x[:, :-k]], axis=1)
        k *= 2
    return x


def _cumsum_rows(x):
    k = 1
    while k < x.shape[0]:
        pad = jnp.zeros_like(x[:k, :])
        x = x + jnp.concatenate([pad, x[:-k, :]], axis=0)
        k *= 2
    return x


def _route_body(lg_ref, q4_ref, wq4_ref, te_ref, aux_ref):
    lg = lg_ref[...]
    lane = jax.lax.broadcasted_iota(jnp.int32, lg.shape, 1)
    valid = lane < NUM_EXPERTS
    neg = jnp.float32(-1e30)
    lg = jnp.where(valid, lg, neg)
    m = jnp.max(lg, axis=-1, keepdims=True)
    e = jnp.where(valid, jnp.exp(lg - m), 0.0)
    p = e / jnp.sum(e, axis=-1, keepdims=True)

    imp = jnp.sum(p, axis=0)
    aux = jnp.sum(imp * imp) / (NT * NT) * NUM_EXPERTS
    aux_ref[0, 0] = aux

    work = p
    w8 = jnp.zeros_like(p)
    selmask = jnp.zeros(p.shape, jnp.int32)
    wsum = jnp.zeros((p.shape[0], 1), jnp.float32)
    picks = []
    for _ in range(TOP_K):
        cur = jnp.max(work, axis=-1, keepdims=True)
        idx = jnp.min(jnp.where(work == cur, lane, LANES), axis=-1,
                      keepdims=True)
        picks.append((idx, cur))
        wsum = wsum + cur
        work = jnp.where(lane == idx, neg, work)
        selmask = selmask + jnp.where(lane == idx, 1, 0)
    inv = 1.0 / (wsum + 1e-20)
    for idx, cur in picks:
        w8 = w8 + jnp.where(lane == idx, cur * inv, 0.0)

    n_e = jnp.sum(selmask, axis=0, keepdims=True)
    rounded = (((n_e + TT - 1) >> 8) << 8)
    off = _cumsum_lanes(rounded, NUM_EXPERTS) - rounded
    erank = _cumsum_rows(selmask) - selmask
    slot = off + erank
    jrank = _cumsum_lanes(selmask, NUM_EXPERTS) - selmask
    sel = selmask > 0
    qcols, wcols = [], []
    for j in range(TOP_K):
        pickj = sel & (jrank == j)
        qcols.append(jnp.sum(jnp.where(pickj, slot, 0), axis=-1,
                             keepdims=True))
        wcols.append(jnp.sum(jnp.where(pickj, w8, 0.0), axis=-1,
                             keepdims=True))
    q4_ref[...] = jnp.concatenate(qcols, axis=-1)
    wq4_ref[...] = jnp.concatenate(wcols, axis=-1)

    kk = jax.lax.broadcasted_iota(jnp.int32, (LANES, LANES), 0) * TT
    ee = jax.lax.broadcasted_iota(jnp.int32, (LANES, LANES), 1)
    cmp = jnp.where((kk >= off[0:1, :]) & (ee < NUM_EXPERTS), 1, 0)
    te_ref[...] = jnp.sum(cmp, axis=-1, keepdims=True) - 1


def _k_route(logits):
    return pl.pallas_call(
        _route_body,
        out_shape=[
            jax.ShapeDtypeStruct((NT, TOP_K), jnp.int32),
            jax.ShapeDtypeStruct((NT, TOP_K), jnp.float32),
            jax.ShapeDtypeStruct((LANES, 1), jnp.int32),
            jax.ShapeDtypeStruct((1, 1), jnp.float32),
        ],
        out_specs=[
            pl.BlockSpec((NT, TOP_K), lambda: (0, 0)),
            pl.BlockSpec((NT, TOP_K), lambda: (0, 0)),
            pl.BlockSpec((LANES, 1), lambda: (0, 0)),
            pl.BlockSpec(memory_space=pltpu.SMEM),
        ],
    )(logits)


_NC16 = APW // 16


def _sc_disp_body(q_hbm, nx_hbm, disp_hbm, qv, rows0, rows1, semg, sems):
    wid = lax.axis_index("s") * 2 + lax.axis_index("c")
    lane = jax.lax.broadcasted_iota(jnp.int32, (16,), 0)
    pltpu.sync_copy(q_hbm.at[wid], qv)
    bufs = (rows0, rows1)

    def gather(c):
        gvec = (wid * APW + c * 16 + lane) >> 2
        return pltpu.async_copy(nx_hbm.at[gvec], bufs[c % 2], semg)

    g = gather(0)
    for c in range(_NC16):
        g_next = gather(c + 1) if c + 1 < _NC16 else None
        g.wait()
        svec = qv[pl.ds(c * 16, 16)]
        pltpu.async_copy(bufs[c % 2], disp_hbm.at[svec], sems).wait()
        g = g_next


def _sc_dispatch(q_flat, nx_f32):
    mesh = plsc.VectorSubcoreMesh(core_axis_name="c", subcore_axis_name="s")
    f = functools.partial(
        pl.kernel,
        out_type=jax.ShapeDtypeStruct((CAP, IN_DIM), jnp.float32),
        mesh=mesh,
        scratch_types=[
            pltpu.VMEM((APW,), jnp.int32),
            pltpu.VMEM((16, IN_DIM), jnp.float32),
            pltpu.VMEM((16, IN_DIM), jnp.float32),
            pltpu.SemaphoreType.DMA,
            pltpu.SemaphoreType.DMA,
        ],
    )(_sc_disp_body)
    return f(q_flat, nx_f32)


_NC2 = TPW // 2


def _sc_comb_body(dout_hbm, q_hbm, wq_hbm, routed_hbm, qv, wv, rows0,
                  rows1, semg):
    wid = lax.axis_index("s") * 2 + lax.axis_index("c")
    pltpu.sync_copy(q_hbm.at[wid], qv)
    bufs = (rows0, rows1)

    def gather(c):
        return pltpu.async_copy(
            dout_hbm.at[qv.at[pl.ds(c * 8, 8)]], bufs[c % 2], semg)

    g = gather(0)
    for c in range(_NC2):
        g_next = gather(c + 1) if c + 1 < _NC2 else None
        pltpu.sync_copy(wq_hbm.at[wid, pl.ds(c * 8, 8)], wv)
        g.wait()
        buf = bufs[c % 2]
        ws = [wv[r, :] for r in range(8)]

        def body(k, _):
            ds = pl.ds(k * 16, 16)
            buf[0, ds] = ((ws[0] * buf[0, ds] + ws[1] * buf[1, ds]) +
                          (ws[2] * buf[2, ds] + ws[3] * buf[3, ds]))
            buf[4, ds] = ((ws[4] * buf[4, ds] + ws[5] * buf[5, ds]) +
                          (ws[6] * buf[6, ds] + ws[7] * buf[7, ds]))
            return 0

        lax.fori_loop(0, OUT_DIM // 16, body, 0)
        t0 = wid * TPW + c * 2
        pltpu.sync_copy(buf.at[pl.ds(0, 1)], routed_hbm.at[pl.ds(t0, 1)])
        pltpu.sync_copy(buf.at[pl.ds(4, 1)],
                        routed_hbm.at[pl.ds(t0 + 1, 1)])
        g = g_next


def _sc_combine(dout_f32, q_flat, wq_flat):
    mesh = plsc.VectorSubcoreMesh(core_axis_name="c", subcore_axis_name="s")
    f = functools.partial(
        pl.kernel,
        out_type=jax.ShapeDtypeStruct((NT, OUT_DIM), jnp.float32),
        mesh=mesh,
        scratch_types=[
            pltpu.VMEM((APW,), jnp.int32),
            pltpu.VMEM((8, 16), jnp.float32),
            pltpu.VMEM((8, OUT_DIM), jnp.float32),
            pltpu.VMEM((8, OUT_DIM), jnp.float32),
            pltpu.SemaphoreType.DMA,
        ],
    )(_sc_comb_body)
    return f(dout_f32, q_flat, wq_flat)


def _k1_body(te_ref, nx_ref, wg_ref, wv_ref, h_ref):
    nx = nx_ref[...].astype(jnp.bfloat16)
    g = jax.lax.dot_general(nx, wg_ref[0], (((1,), (1,)), ((), ())),
                            preferred_element_type=jnp.float32)
    v = jax.lax.dot_general(nx, wv_ref[0], (((1,), (1,)), ((), ())),
                            preferred_element_type=jnp.float32)
    h_ref[...] = (g * jax.lax.logistic(g) * v).astype(jnp.bfloat16)


def _k_gmm1(te, disp, ew12):
    bh = 1024
    nh = ROUTED_HIDDEN // bh
    return pl.pallas_call(
        _k1_body,
        grid_spec=pltpu.PrefetchScalarGridSpec(
            num_scalar_prefetch=1,
            grid=(nh, NTILE),
            in_specs=[
                pl.BlockSpec((TT, IN_DIM), lambda c, i, te: (i, 0)),
                pl.BlockSpec((1, bh, IN_DIM), lambda c, i, te: (te[i], c, 0)),
                pl.BlockSpec((1, bh, IN_DIM),
                             lambda c, i, te: (te[i], nh + c, 0)),
            ],
            out_specs=pl.BlockSpec((TT, bh), lambda c, i, te: (i, c)),
        ),
        out_shape=jax.ShapeDtypeStruct((CAP, ROUTED_HIDDEN), jnp.bfloat16),
    )(te, disp, ew12, ew12)


def _k2_body(te_ref, h_ref, w3_ref, o_ref):
    o_ref[...] = jax.lax.dot_general(
        h_ref[...], w3_ref[0], (((1,), (1,)), ((), ())),
        preferred_element_type=jnp.float32)


def _k_gmm2(te, hb, ew3):
    bo = 1024
    return pl.pallas_call(
        _k2_body,
        grid_spec=pltpu.PrefetchScalarGridSpec(
            num_scalar_prefetch=1,
            grid=(OUT_DIM // bo, NTILE),
            in_specs=[
                pl.BlockSpec((TT, ROUTED_HIDDEN), lambda c, i, te: (i, 0)),
                pl.BlockSpec((1, bo, ROUTED_HIDDEN),
                             lambda c, i, te: (te[i], c, 0)),
            ],
            out_specs=pl.BlockSpec((TT, bo), lambda c, i, te: (i, c)),
        ),
        out_shape=jax.ShapeDtypeStruct((CAP, OUT_DIM), jnp.float32),
    )(te, hb, ew3)


def _s1_body(nx_ref, wg_ref, wv_ref, h_ref):
    nx = nx_ref[...].astype(jnp.bfloat16)
    g = jax.lax.dot_general(nx, wg_ref[...], (((1,), (1,)), ((), ())),
                            preferred_element_type=jnp.float32)
    v = jax.lax.dot_general(nx, wv_ref[...], (((1,), (1,)), ((), ())),
                            preferred_element_type=jnp.float32)
    h_ref[...] = (g * jax.lax.logistic(g) * v).astype(jnp.bfloat16)


def _k_s1(nx, w12):
    rt, bh = 8, 1024
    bt = NT // rt
    return pl.pallas_call(
        _s1_body,
        grid=(SHARED_HIDDEN // bh, rt),
        in_specs=[
            pl.BlockSpec((bt, IN_DIM), lambda c, i: (i, 0)),
            pl.BlockSpec((bh, IN_DIM), lambda c, i: (c, 0)),
            pl.BlockSpec((bh, IN_DIM),
                         lambda c, i: (SHARED_HIDDEN // bh + c, 0)),
        ],
        out_specs=pl.BlockSpec((bt, bh), lambda c, i: (i, c)),
        out_shape=jax.ShapeDtypeStruct((NT, SHARED_HIDDEN), jnp.bfloat16),
    )(nx, w12, w12)


def _final_body(r_ref, hs_ref, w3_ref, wpost_ref, o_ref):
    sh = jax.lax.dot_general(
        hs_ref[...], w3_ref[...], (((1,), (1,)), ((), ())),
        preferred_element_type=jnp.float32)
    y = sh + r_ref[...]
    var = jnp.mean(y * y, axis=-1, keepdims=True)
    y = y * jax.lax.rsqrt(var + EPS) * wpost_ref[...]
    o_ref[...] = jnp.clip(y, -30.0, 30.0)


def _k_final(routed, hs, sw3, ln_post_w):
    rt = 16
    bt = NT // rt
    return pl.pallas_call(
        _final_body,
        grid=(rt,),
        in_specs=[
            pl.BlockSpec((bt, OUT_DIM), lambda i: (i, 0)),
            pl.BlockSpec((bt, SHARED_HIDDEN), lambda i: (i, 0)),
            pl.BlockSpec((OUT_DIM, SHARED_HIDDEN), lambda i: (0, 0)),
            pl.BlockSpec((1, OUT_DIM), lambda i: (0, 0)),
        ],
        out_specs=pl.BlockSpec((bt, OUT_DIM), lambda i: (i, 0)),
        out_shape=jax.ShapeDtypeStruct((NT, OUT_DIM), jnp.float32),
    )(routed, hs, sw3, ln_post_w.reshape(1, OUT_DIM))


def kernel(x, ln_pre_w, router_w, shared_w12, shared_w3, expert_w12,
           expert_w3, ln_post_w):
    B, S, D = x.shape
    tokens = x.reshape(B * S // K, D * K)

    router_w_pad = jnp.zeros((LANES, IN_DIM), jnp.float32).at[:NUM_EXPERTS].set(
        router_w)
    nx, logits = _k_norm(tokens, ln_pre_w, router_w_pad)
    q4, wq4, te2d, aux = _k_route(logits)
    te = te2d.reshape(LANES)[:80]
    q_flat = q4.reshape(NW, APW)
    wq_flat = jnp.broadcast_to(wq4.reshape(NW, APW, 1), (NW, APW, 16))

    disp = _sc_dispatch(q_flat, nx)

    ew12 = expert_w12.astype(jnp.bfloat16)
    ew3 = expert_w3.astype(jnp.bfloat16)
    hb = _k_gmm1(te, disp, ew12)
    dout = _k_gmm2(te, hb, ew3)

    routed = _sc_combine(dout, q_flat, wq_flat)
    hs = _k_s1(nx, shared_w12.astype(jnp.bfloat16))
    final = _k_final(routed, hs, shared_w3.astype(jnp.bfloat16), ln_post_w)
    return final.reshape(B, S // K, OUT_DIM), aux[0, 0]


# speedup vs baseline: 3.1464x; 1.0345x over previous
import functools

import jax
import jax.numpy as jnp
from jax import lax
from jax.experimental import pallas as pl
from jax.experimental.pallas import tpu as pltpu
from jax.experimental.pallas import tpu_sc as plsc

ENC_DIM = 1280
K = 2
IN_DIM = ENC_DIM * K
OUT_DIM = 4096
NUM_EXPERTS = 8
TOP_K = 4
ROUTED_HIDDEN = 2048
SHARED_HIDDEN = 2048
EPS = 1e-6
NT = 4096
LANES = 128

TT = 512
NTILE = 40
CAP = NTILE * TT
NA = NT * TOP_K
NW = 32
TPW = NT // NW
APW = TPW * TOP_K


def _norm_body(x_ref, w_ref, rw_ref, nx_ref, lg_ref):
    x = x_ref[...]
    var = jnp.mean(x * x, axis=-1, keepdims=True)
    nx = x * jax.lax.rsqrt(var + EPS) * w_ref[...]
    nx_ref[...] = nx
    lg_ref[...] = jax.lax.dot_general(
        nx, rw_ref[...], (((1,), (1,)), ((), ())),
        preferred_element_type=jnp.float32)


def _k_norm(tokens, ln_pre_w, router_w_pad):
    rt = 16
    bt = NT // rt
    return pl.pallas_call(
        _norm_body,
        grid=(rt,),
        in_specs=[
            pl.BlockSpec((bt, IN_DIM), lambda i: (i, 0)),
            pl.BlockSpec((1, IN_DIM), lambda i: (0, 0)),
            pl.BlockSpec((LANES, IN_DIM), lambda i: (0, 0)),
        ],
        out_specs=[
            pl.BlockSpec((bt, IN_DIM), lambda i: (i, 0)),
            pl.BlockSpec((bt, LANES), lambda i: (i, 0)),
        ],
        out_shape=[
            jax.ShapeDtypeStruct((NT, IN_DIM), jnp.float32),
            jax.ShapeDtypeStruct((NT, LANES), jnp.float32),
        ],
    )(tokens, ln_pre_w.reshape(1, IN_DIM), router_w_pad)


def _cumsum_lanes(x, width):
    k = 1
    while k < width:
        pad = jnp.zeros_like(x[:, :k])
        x = x + jnp.concatenate([pad, x[:, :-k]], axis=1)
        k *= 2
    return x


def _cumsum_rows(x):
    k = 1
    while k < x.shape[0]:
        pad = jnp.zeros_like(x[:k, :])
        x = x + jnp.concatenate([pad, x[:-k, :]], axis=0)
        k *= 2
    return x


def _route_body(lg_ref, q4_ref, wq4_ref, te_ref, aux_ref):
    lg = lg_ref[...]
    lane = jax.lax.broadcasted_iota(jnp.int32, lg.shape, 1)
    valid = lane < NUM_EXPERTS
    neg = jnp.float32(-1e30)
    lg = jnp.where(valid, lg, neg)
    m = jnp.max(lg, axis=-1, keepdims=True)
    e = jnp.where(valid, jnp.exp(lg - m), 0.0)
    p = e / jnp.sum(e, axis=-1, keepdims=True)

    imp = jnp.sum(p, axis=0)
    aux = jnp.sum(imp * imp) / (NT * NT) * NUM_EXPERTS
    aux_ref[0, 0] = aux

    work = p
    w8 = jnp.zeros_like(p)
    selmask = jnp.zeros(p.shape, jnp.int32)
    wsum = jnp.zeros((p.shape[0], 1), jnp.float32)
    picks = []
    for _ in range(TOP_K):
        cur = jnp.max(work, axis=-1, keepdims=True)
        idx = jnp.min(jnp.where(work == cur, lane, LANES), axis=-1,
                      keepdims=True)
        picks.append((idx, cur))
        wsum = wsum + cur
        work = jnp.where(lane == idx, neg, work)
        selmask = selmask + jnp.where(lane == idx, 1, 0)
    inv = 1.0 / (wsum + 1e-20)
    for idx, cur in picks:
        w8 = w8 + jnp.where(lane == idx, cur * inv, 0.0)

    n_e = jnp.sum(selmask, axis=0, keepdims=True)
    _sh = TT.bit_length() - 1
    rounded = (((n_e + TT - 1) >> _sh) << _sh)
    off = _cumsum_lanes(rounded, NUM_EXPERTS) - rounded
    erank = _cumsum_rows(selmask) - selmask
    slot = off + erank
    jrank = _cumsum_lanes(selmask, NUM_EXPERTS) - selmask
    sel = selmask > 0
    qcols, wcols = [], []
    for j in range(TOP_K):
        pickj = sel & (jrank == j)
        qcols.append(jnp.sum(jnp.where(pickj, slot, 0), axis=-1,
                             keepdims=True))
        wcols.append(jnp.sum(jnp.where(pickj, w8, 0.0), axis=-1,
                             keepdims=True))
    q4_ref[...] = jnp.concatenate(qcols, axis=-1)
    wq4_ref[...] = jnp.concatenate(wcols, axis=-1)

    kk = jax.lax.broadcasted_iota(jnp.int32, (LANES, LANES), 0) * TT
    ee = jax.lax.broadcasted_iota(jnp.int32, (LANES, LANES), 1)
    cmp = jnp.where((kk >= off[0:1, :]) & (ee < NUM_EXPERTS), 1, 0)
    te_ref[...] = jnp.sum(cmp, axis=-1, keepdims=True) - 1


def _k_route(logits):
    return pl.pallas_call(
        _route_body,
        out_shape=[
            jax.ShapeDtypeStruct((NT, TOP_K), jnp.int32),
            jax.ShapeDtypeStruct((NT, TOP_K), jnp.float32),
            jax.ShapeDtypeStruct((LANES, 1), jnp.int32),
            jax.ShapeDtypeStruct((1, 1), jnp.float32),
        ],
        out_specs=[
            pl.BlockSpec((NT, TOP_K), lambda: (0, 0)),
            pl.BlockSpec((NT, TOP_K), lambda: (0, 0)),
            pl.BlockSpec((LANES, 1), lambda: (0, 0)),
            pl.BlockSpec(memory_space=pltpu.SMEM),
        ],
    )(logits)


_NC16 = APW // 16


def _sc_disp_body(q_hbm, nx_hbm, disp_hbm, qv, rows0, rows1, semg, sems):
    wid = lax.axis_index("s") * 2 + lax.axis_index("c")
    lane = jax.lax.broadcasted_iota(jnp.int32, (16,), 0)
    pltpu.sync_copy(q_hbm.at[wid], qv)
    bufs = (rows0, rows1)

    def gather(c):
        gvec = (wid * APW + c * 16 + lane) >> 2
        return pltpu.async_copy(nx_hbm.at[gvec], bufs[c % 2], semg)

    g = gather(0)
    for c in range(_NC16):
        g_next = gather(c + 1) if c + 1 < _NC16 else None
        g.wait()
        svec = qv[pl.ds(c * 16, 16)]
        pltpu.async_copy(bufs[c % 2], disp_hbm.at[svec], sems).wait()
        g = g_next


def _sc_dispatch(q_flat, nx_f32):
    mesh = plsc.VectorSubcoreMesh(core_axis_name="c", subcore_axis_name="s")
    f = functools.partial(
        pl.kernel,
        out_type=jax.ShapeDtypeStruct((CAP, IN_DIM), jnp.float32),
        mesh=mesh,
        scratch_types=[
            pltpu.VMEM((APW,), jnp.int32),
            pltpu.VMEM((16, IN_DIM), jnp.float32),
            pltpu.VMEM((16, IN_DIM), jnp.float32),
            pltpu.SemaphoreType.DMA,
            pltpu.SemaphoreType.DMA,
        ],
    )(_sc_disp_body)
    return f(q_flat, nx_f32)


_NC2 = TPW // 2


def _sc_comb_body(dout_hbm, q_hbm, wq_hbm, routed_hbm, qv, wv, rows0,
                  rows1, semg):
    wid = lax.axis_index("s") * 2 + lax.axis_index("c")
    pltpu.sync_copy(q_hbm.at[wid], qv)
    bufs = (rows0, rows1)

    def gather(c):
        return pltpu.async_copy(
            dout_hbm.at[qv.at[pl.ds(c * 8, 8)]], bufs[c % 2], semg)

    g = gather(0)
    for c in range(_NC2):
        g_next = gather(c + 1) if c + 1 < _NC2 else None
        pltpu.sync_copy(wq_hbm.at[wid, pl.ds(c * 8, 8)], wv)
        g.wait()
        buf = bufs[c % 2]
        ws = [wv[r, :] for r in range(8)]

        def body(k, _):
            ds = pl.ds(k * 16, 16)
            buf[0, ds] = ((ws[0] * buf[0, ds] + ws[1] * buf[1, ds]) +
                          (ws[2] * buf[2, ds] + ws[3] * buf[3, ds]))
            buf[4, ds] = ((ws[4] * buf[4, ds] + ws[5] * buf[5, ds]) +
                          (ws[6] * buf[6, ds] + ws[7] * buf[7, ds]))
            return 0

        lax.fori_loop(0, OUT_DIM // 16, body, 0)
        t0 = wid * TPW + c * 2
        pltpu.sync_copy(buf.at[pl.ds(0, 1)], routed_hbm.at[pl.ds(t0, 1)])
        pltpu.sync_copy(buf.at[pl.ds(4, 1)],
                        routed_hbm.at[pl.ds(t0 + 1, 1)])
        g = g_next


def _sc_combine(dout_f32, q_flat, wq_flat):
    mesh = plsc.VectorSubcoreMesh(core_axis_name="c", subcore_axis_name="s")
    f = functools.partial(
        pl.kernel,
        out_type=jax.ShapeDtypeStruct((NT, OUT_DIM), jnp.float32),
        mesh=mesh,
        scratch_types=[
            pltpu.VMEM((APW,), jnp.int32),
            pltpu.VMEM((8, 16), jnp.float32),
            pltpu.VMEM((8, OUT_DIM), jnp.float32),
            pltpu.VMEM((8, OUT_DIM), jnp.float32),
            pltpu.SemaphoreType.DMA,
        ],
    )(_sc_comb_body)
    return f(dout_f32, q_flat, wq_flat)


def _k1_body(te_ref, nx_ref, wg_ref, wv_ref, h_ref):
    nx = nx_ref[...].astype(jnp.bfloat16)
    g = jax.lax.dot_general(nx, wg_ref[0], (((1,), (1,)), ((), ())),
                            preferred_element_type=jnp.float32)
    v = jax.lax.dot_general(nx, wv_ref[0], (((1,), (1,)), ((), ())),
                            preferred_element_type=jnp.float32)
    h_ref[...] = (g * jax.lax.logistic(g) * v).astype(jnp.bfloat16)


def _k_gmm1(te, disp, ew12):
    bh = 1024
    nh = ROUTED_HIDDEN // bh
    return pl.pallas_call(
        _k1_body,
        grid_spec=pltpu.PrefetchScalarGridSpec(
            num_scalar_prefetch=1,
            grid=(nh, NTILE),
            in_specs=[
                pl.BlockSpec((TT, IN_DIM), lambda c, i, te: (i, 0)),
                pl.BlockSpec((1, bh, IN_DIM), lambda c, i, te: (te[i], c, 0)),
                pl.BlockSpec((1, bh, IN_DIM),
                             lambda c, i, te: (te[i], nh + c, 0)),
            ],
            out_specs=pl.BlockSpec((TT, bh), lambda c, i, te: (i, c)),
        ),
        out_shape=jax.ShapeDtypeStruct((CAP, ROUTED_HIDDEN), jnp.bfloat16),
    )(te, disp, ew12, ew12)


def _k2_body(te_ref, h_ref, w3_ref, o_ref):
    o_ref[...] = jax.lax.dot_general(
        h_ref[...], w3_ref[0], (((1,), (1,)), ((), ())),
        preferred_element_type=jnp.float32)


def _k_gmm2(te, hb, ew3):
    bo = 1024
    return pl.pallas_call(
        _k2_body,
        grid_spec=pltpu.PrefetchScalarGridSpec(
            num_scalar_prefetch=1,
            grid=(OUT_DIM // bo, NTILE),
            in_specs=[
                pl.BlockSpec((TT, ROUTED_HIDDEN), lambda c, i, te: (i, 0)),
                pl.BlockSpec((1, bo, ROUTED_HIDDEN),
                             lambda c, i, te: (te[i], c, 0)),
            ],
            out_specs=pl.BlockSpec((TT, bo), lambda c, i, te: (i, c)),
        ),
        out_shape=jax.ShapeDtypeStruct((CAP, OUT_DIM), jnp.float32),
    )(te, hb, ew3)


def _s1_body(nx_ref, wg_ref, wv_ref, h_ref):
    nx = nx_ref[...].astype(jnp.bfloat16)
    g = jax.lax.dot_general(nx, wg_ref[...], (((1,), (1,)), ((), ())),
                            preferred_element_type=jnp.float32)
    v = jax.lax.dot_general(nx, wv_ref[...], (((1,), (1,)), ((), ())),
                            preferred_element_type=jnp.float32)
    h_ref[...] = (g * jax.lax.logistic(g) * v).astype(jnp.bfloat16)


def _k_s1(nx, w12):
    rt, bh = 8, 1024
    bt = NT // rt
    return pl.pallas_call(
        _s1_body,
        grid=(SHARED_HIDDEN // bh, rt),
        in_specs=[
            pl.BlockSpec((bt, IN_DIM), lambda c, i: (i, 0)),
            pl.BlockSpec((bh, IN_DIM), lambda c, i: (c, 0)),
            pl.BlockSpec((bh, IN_DIM),
                         lambda c, i: (SHARED_HIDDEN // bh + c, 0)),
        ],
        out_specs=pl.BlockSpec((bt, bh), lambda c, i: (i, c)),
        out_shape=jax.ShapeDtypeStruct((NT, SHARED_HIDDEN), jnp.bfloat16),
    )(nx, w12, w12)


def _final_body(r_ref, hs_ref, w3_ref, wpost_ref, o_ref):
    sh = jax.lax.dot_general(
        hs_ref[...], w3_ref[...], (((1,), (1,)), ((), ())),
        preferred_element_type=jnp.float32)
    y = sh + r_ref[...]
    var = jnp.mean(y * y, axis=-1, keepdims=True)
    y = y * jax.lax.rsqrt(var + EPS) * wpost_ref[...]
    o_ref[...] = jnp.clip(y, -30.0, 30.0)


def _k_final(routed, hs, sw3, ln_post_w):
    rt = 16
    bt = NT // rt
    return pl.pallas_call(
        _final_body,
        grid=(rt,),
        in_specs=[
            pl.BlockSpec((bt, OUT_DIM), lambda i: (i, 0)),
            pl.BlockSpec((bt, SHARED_HIDDEN), lambda i: (i, 0)),
            pl.BlockSpec((OUT_DIM, SHARED_HIDDEN), lambda i: (0, 0)),
            pl.BlockSpec((1, OUT_DIM), lambda i: (0, 0)),
        ],
        out_specs=pl.BlockSpec((bt, OUT_DIM), lambda i: (i, 0)),
        out_shape=jax.ShapeDtypeStruct((NT, OUT_DIM), jnp.float32),
    )(routed, hs, sw3, ln_post_w.reshape(1, OUT_DIM))


def kernel(x, ln_pre_w, router_w, shared_w12, shared_w3, expert_w12,
           expert_w3, ln_post_w):
    B, S, D = x.shape
    tokens = x.reshape(B * S // K, D * K)

    router_w_pad = jnp.zeros((LANES, IN_DIM), jnp.float32).at[:NUM_EXPERTS].set(
        router_w)
    nx, logits = _k_norm(tokens, ln_pre_w, router_w_pad)
    q4, wq4, te2d, aux = _k_route(logits)
    te = te2d.reshape(LANES)[:80]
    q_flat = q4.reshape(NW, APW)
    wq_flat = jnp.broadcast_to(wq4.reshape(NW, APW, 1), (NW, APW, 16))

    disp = _sc_dispatch(q_flat, nx)

    ew12 = expert_w12.astype(jnp.bfloat16)
    ew3 = expert_w3.astype(jnp.bfloat16)
    hb = _k_gmm1(te, disp, ew12)
    dout = _k_gmm2(te, hb, ew3)

    routed = _sc_combine(dout, q_flat, wq_flat)
    hs = _k_s1(nx, shared_w12.astype(jnp.bfloat16))
    final = _k_final(routed, hs, shared_w3.astype(jnp.bfloat16), ln_post_w)
    return final.reshape(B, S // K, OUT_DIM), aux[0, 0]
